# baseline (speedup 1.0000x reference)
import jax
import jax.numpy as jnp
from jax import lax
from jax.experimental import pallas as pl
from jax.experimental.pallas import tpu as pltpu
from jax.experimental.pallas import tpu_sc as plsc

P = 196
NW = 32


def _body(patches_hbm, *out_refs):
    cid = lax.axis_index("c")
    sid = lax.axis_index("s")
    wid = sid * 2 + cid

    for i in range(P):
        @pl.when(wid == (i % NW))
        def _(i=i):
            out = out_refs[i]
            if i > 0:
                pltpu.sync_copy(patches_hbm.at[pl.ds(0, i)],
                                out.at[pl.ds(0, i)])
            if i < P - 1:
                pltpu.sync_copy(patches_hbm.at[pl.ds(i + 1, P - 1 - i)],
                                out.at[pl.ds(i, P - 1 - i)])


_mesh = plsc.VectorSubcoreMesh(core_axis_name="c", subcore_axis_name="s")
_out_type = tuple(jax.ShapeDtypeStruct((P - 1, 4, 192), jnp.float32)
                  for _ in range(P))
_sc_call = pl.kernel(_body, out_type=_out_type, mesh=_mesh)


def kernel(patches):
    return _sc_call(patches)


# speedup vs baseline: 13.5009x; 13.5009x over previous
import jax
import jax.numpy as jnp
from jax import lax
from jax.experimental import pallas as pl
from jax.experimental.pallas import tpu as pltpu
from jax.experimental.pallas import tpu_sc as plsc

P = 196
NW = 32
HALF = 98


def _body(patches_hbm, *rest):
    outs = rest[:P]
    buf = rest[P]
    sem = rest[P + 1]
    cid = lax.axis_index("c")
    sid = lax.axis_index("s")
    wid = sid * 2 + cid

    for (a, b) in ((0, HALF), (HALF, P)):
        pltpu.sync_copy(patches_hbm.at[pl.ds(a, b - a)], buf)
        for w in range(NW):
            @pl.when(wid == w)
            def _(w=w, a=a, b=b):
                handles = []
                for i in range(w, P, NW):
                    out = outs[i]
                    if i < a:
                        handles.append(pltpu.async_copy(
                            buf, out.at[pl.ds(a - 1, b - a)], sem))
                    elif i >= b:
                        handles.append(pltpu.async_copy(
                            buf, out.at[pl.ds(a, b - a)], sem))
                    else:
                        n1 = i - a
                        if n1 > 0:
                            handles.append(pltpu.async_copy(
                                buf.at[pl.ds(0, n1)],
                                out.at[pl.ds(a, n1)], sem))
                        n2 = b - 1 - i
                        if n2 > 0:
                            handles.append(pltpu.async_copy(
                                buf.at[pl.ds(n1 + 1, n2)],
                                out.at[pl.ds(i, n2)], sem))
                for h in handles:
                    h.wait()


_mesh = plsc.VectorSubcoreMesh(core_axis_name="c", subcore_axis_name="s")
_out_type = tuple(jax.ShapeDtypeStruct((P - 1, 4, 192), jnp.float32)
                  for _ in range(P))
_sc_call = pl.kernel(
    _body, out_type=_out_type, mesh=_mesh,
    scratch_types=[pltpu.VMEM((HALF, 4, 192), jnp.float32),
                   pltpu.SemaphoreType.DMA],
)


def kernel(patches):
    return _sc_call(patches)


# speedup vs baseline: 64.5043x; 4.7778x over previous
import jax
import jax.numpy as jnp
from jax import lax
from jax.experimental import pallas as pl
from jax.experimental.pallas import tpu as pltpu

P = 196
K = 28


def _make_body(base):
    def _body(in_ref, *out_refs):
        a = in_ref[:, :, 0:P - 1]
        b = in_ref[:, :, 1:P]
        lane = lax.broadcasted_iota(jnp.int32, (4, 192, P - 1), 2)
        for k in range(K):
            i = base + k
            out_refs[k][...] = jnp.where(lane < i, a, b)
    return _body


def _group_call(base):
    return pl.pallas_call(
        _make_body(base),
        out_shape=tuple(jax.ShapeDtypeStruct((4, 192, P - 1), jnp.float32)
                        for _ in range(K)),
    )


def kernel(patches):
    pt = jnp.transpose(patches, (1, 2, 0))
    outs = []
    for base in range(0, P, K):
        outs.extend(_group_call(base)(pt))
    return tuple(jnp.transpose(o, (2, 0, 1)) for o in outs)


# speedup vs baseline: 69.9294x; 1.0841x over previous
import jax
import jax.numpy as jnp
from jax import lax
from jax.experimental import pallas as pl
from jax.experimental.pallas import tpu as pltpu

P = 196
K = 49


def _make_body(base):
    def _body(in_ref, *out_refs):
        a = in_ref[:, :, 0:P - 1]
        b = in_ref[:, :, 1:P]
        lane = lax.broadcasted_iota(jnp.int32, (4, 192, P - 1), 2)
        for k in range(K):
            i = base + k
            out_refs[k][...] = jnp.where(lane < i, a, b)
    return _body


def _group_call(base):
    return pl.pallas_call(
        _make_body(base),
        out_shape=tuple(jax.ShapeDtypeStruct((4, 192, P - 1), jnp.float32)
                        for _ in range(K)),
    )


def kernel(patches):
    pt = jnp.transpose(patches, (1, 2, 0))
    outs = []
    for base in range(0, P, K):
        outs.extend(_group_call(base)(pt))
    return tuple(jnp.transpose(o, (2, 0, 1)) for o in outs)
